# baseline (speedup 1.0000x reference)
import functools

import jax
import jax.numpy as jnp
from jax import lax
from jax.experimental import pallas as pl
from jax.experimental.pallas import tpu as pltpu
from jax.experimental.pallas import tpu_sc as plsc

_V = 100000
_E = 64
_H = 2048
_IE = 32
_N = 8192
_NW = 32
_TPW = _N // _NW
_NCH = _TPW // 16
_TILE = 64
_NS = 12288
_NT = _NS // _TILE


def _wid():
    nc = plsc.get_sparse_core_info().num_cores
    return lax.axis_index("s") * nc + lax.axis_index("c")


def _route_body(tids_hbm, counts_hbm, eid_hbm, rank_hbm,
                tid_v, eid_v, rank_v, cnt_v):
    wid = _wid()
    base = wid * _TPW
    iota = lax.iota(jnp.int32, 16)
    pltpu.sync_copy(tids_hbm.at[pl.ds(base, _TPW)], tid_v)
    for c in range(_NCH):
        t = tid_v[pl.ds(c * 16, 16)]
        t = jnp.minimum(jnp.maximum(t, 0), _V - 1)
        eid_v[pl.ds(c * 16, 16)] = lax.rem(t, _E)
        rank_v[pl.ds(c * 16, 16)] = jnp.zeros((16,), jnp.int32)

    def e_body(e, carry_unused):
        ev = jnp.zeros((16,), jnp.int32) + e
        carry = jnp.int32(0)
        for c in range(_NCH):
            ee = eid_v[pl.ds(c * 16, 16)]
            m = ee == ev
            mm = m.astype(jnp.int32)
            exc = plsc.cumsum(mm) - mm
            old = rank_v[pl.ds(c * 16, 16)]
            rank_v[pl.ds(c * 16, 16)] = jnp.where(m, exc + carry, old)
            carry = carry + jnp.sum(mm)
        plsc.store_scatter(cnt_v, [ev], jnp.zeros((16,), jnp.int32) + carry,
                           mask=iota == 0)
        return carry_unused

    lax.fori_loop(0, _E, e_body, jnp.int32(0))
    pltpu.sync_copy(cnt_v, counts_hbm.at[wid])
    pltpu.sync_copy(eid_v, eid_hbm.at[wid])
    pltpu.sync_copy(rank_v, rank_hbm.at[wid])


def _dispatch_body(x_hbm, counts_hbm, eid_hbm, rank_hbm,
                   xs_hbm, pos_hbm, te_hbm, nact_hbm,
                   allcnt_v, start_v, eid_v, rank_v, slots_v, te_v, nact_v,
                   rowbuf_v, gs0, gs1, gs2, ss0, ss1, ss2):
    wid = _wid()
    base = wid * _TPW
    iota = lax.iota(jnp.int32, 16)
    pltpu.sync_copy(counts_hbm, allcnt_v)

    tot, myb = [], []
    for ec in range(_E // 16):
        t_acc = jnp.zeros((16,), jnp.int32)
        m_acc = jnp.zeros((16,), jnp.int32)
        for w in range(_NW):
            row = allcnt_v[w, pl.ds(ec * 16, 16)]
            t_acc = t_acc + row
            m_acc = m_acc + row * (wid > w).astype(jnp.int32)
        tot.append(t_acc)
        myb.append(m_acc)

    carry = jnp.int32(0)
    pend = []
    for ec in range(_E // 16):
        pc = (tot[ec] + (_TILE - 1)) & ~(_TILE - 1)
        exc = plsc.cumsum(pc) - pc
        ps = exc + carry
        pend.append(ps + pc)
        carry = carry + jnp.sum(pc)
        start_v[pl.ds(ec * 16, 16)] = ps + myb[ec]

    pltpu.sync_copy(eid_hbm.at[wid], eid_v)
    pltpu.sync_copy(rank_hbm.at[wid], rank_v)
    for c in range(_NCH):
        ee = eid_v[pl.ds(c * 16, 16)]
        st = plsc.load_gather(start_v, [ee])
        slots_v[c] = st + rank_v[pl.ds(c * 16, 16)]
    pltpu.sync_copy(slots_v, pos_hbm.at[wid])

    gsems = [gs0, gs1, gs2]
    ssems = [ss0, ss1, ss2]
    hg = [None] * _NCH
    hs = [None] * _NCH
    for c in range(3):
        hg[c] = pltpu.async_copy(x_hbm.at[pl.ds(base + c * 16, 16)],
                                 rowbuf_v.at[c], gsems[c])
    for c in range(_NCH):
        bslot = c % 3
        hg[c].wait()
        hs[c] = pltpu.async_copy(rowbuf_v.at[bslot], xs_hbm.at[slots_v.at[c]],
                                 ssems[bslot])
        nxt = c + 3
        if nxt < _NCH:
            hs[c].wait()
            hg[nxt] = pltpu.async_copy(x_hbm.at[pl.ds(base + nxt * 16, 16)],
                                       rowbuf_v.at[bslot], gsems[bslot])
    for c in range(_NCH - 3, _NCH):
        hs[c].wait()

    @pl.when(wid == 0)
    def _():
        ends = []
        for e in range(_E):
            ch, ln = e // 16, e % 16
            ends.append(jnp.sum(jnp.where(iota == ln, pend[ch], 0)))
        for tc in range(_NT // 16):
            tvec = (iota + tc * 16) * _TILE
            acc = jnp.zeros((16,), jnp.int32)
            for e in range(_E):
                acc = acc + (tvec >= ends[e]).astype(jnp.int32)
            te_v[pl.ds(tc * 16, 16)] = jnp.minimum(acc, _E - 1)
        pltpu.sync_copy(te_v, te_hbm)
        nact_v[...] = jnp.zeros((16,), jnp.int32) + lax.div(carry, _TILE)
        pltpu.sync_copy(nact_v, nact_hbm)


def _combine_body(ys_hbm, pos_hbm, out_hbm, pos_v, rowbuf_v,
                  gs0, gs1, gs2, ss0, ss1, ss2):
    wid = _wid()
    base = wid * _TPW
    pltpu.sync_copy(pos_hbm.at[wid], pos_v)
    gsems = [gs0, gs1, gs2]
    ssems = [ss0, ss1, ss2]
    hg = [None] * _NCH
    hs = [None] * _NCH
    for c in range(3):
        hg[c] = pltpu.async_copy(ys_hbm.at[pos_v.at[c]], rowbuf_v.at[c],
                                 gsems[c])
    for c in range(_NCH):
        bslot = c % 3
        hg[c].wait()
        hs[c] = pltpu.async_copy(rowbuf_v.at[bslot],
                                 out_hbm.at[pl.ds(base + c * 16, 16)],
                                 ssems[bslot])
        nxt = c + 3
        if nxt < _NCH:
            hs[c].wait()
            hg[nxt] = pltpu.async_copy(ys_hbm.at[pos_v.at[nxt]],
                                       rowbuf_v.at[bslot], gsems[bslot])
    for c in range(_NCH - 3, _NCH):
        hs[c].wait()


def _tobf16_body(x_ref, o_ref):
    o_ref[...] = x_ref[...].astype(jnp.bfloat16)


def _gemm_body(te_ref, na_ref, x_ref, g_ref, u_ref, d_ref, o_ref):
    x = x_ref[...].astype(jnp.bfloat16)
    g = g_ref[0]
    u = u_ref[0]
    dn = d_ref[0]
    xg = lax.dot_general(x, g, (((1,), (1,)), ((), ())),
                         preferred_element_type=jnp.float32)
    xu = lax.dot_general(x, u, (((1,), (1,)), ((), ())),
                         preferred_element_type=jnp.float32)
    h = (xg * jax.nn.sigmoid(xg) * xu).astype(jnp.bfloat16)
    o_ref[...] = lax.dot_general(h, dn, (((1,), (1,)), ((), ())),
                                 preferred_element_type=jnp.float32)


_sc_mesh = plsc.VectorSubcoreMesh(core_axis_name="c", subcore_axis_name="s")
_sc_params = pltpu.CompilerParams(needs_layout_passes=False)

_route = pl.kernel(
    _route_body,
    out_type=(jax.ShapeDtypeStruct((_NW, _E), jnp.int32),
              jax.ShapeDtypeStruct((_NW, _TPW), jnp.int32),
              jax.ShapeDtypeStruct((_NW, _TPW), jnp.int32)),
    mesh=_sc_mesh,
    compiler_params=_sc_params,
    scratch_types=[pltpu.VMEM((_TPW,), jnp.int32),
                   pltpu.VMEM((_TPW,), jnp.int32),
                   pltpu.VMEM((_TPW,), jnp.int32),
                   pltpu.VMEM((_E,), jnp.int32)],
)

_tobf16 = pl.pallas_call(
    _tobf16_body,
    grid=(16,),
    in_specs=[pl.BlockSpec((_N // 16, _H), lambda t: (t, 0))],
    out_specs=pl.BlockSpec((_N // 16, _H), lambda t: (t, 0)),
    out_shape=jax.ShapeDtypeStruct((_N, _H), jnp.bfloat16),
)

_dispatch = pl.kernel(
    _dispatch_body,
    out_type=(jax.ShapeDtypeStruct((_NS, _H), jnp.float32),
              jax.ShapeDtypeStruct((_NW, _NCH, 16), jnp.int32),
              jax.ShapeDtypeStruct((_NT,), jnp.int32),
              jax.ShapeDtypeStruct((16,), jnp.int32)),
    mesh=_sc_mesh,
    compiler_params=_sc_params,
    scratch_types=[pltpu.VMEM((_NW, _E), jnp.int32),
                   pltpu.VMEM((_E,), jnp.int32),
                   pltpu.VMEM((_TPW,), jnp.int32),
                   pltpu.VMEM((_TPW,), jnp.int32),
                   pltpu.VMEM((_NCH, 16), jnp.int32),
                   pltpu.VMEM((_NT,), jnp.int32),
                   pltpu.VMEM((16,), jnp.int32),
                   pltpu.VMEM((3, 16, _H), jnp.float32),
                   pltpu.SemaphoreType.DMA, pltpu.SemaphoreType.DMA,
                   pltpu.SemaphoreType.DMA, pltpu.SemaphoreType.DMA,
                   pltpu.SemaphoreType.DMA, pltpu.SemaphoreType.DMA],
)

_combine = pl.kernel(
    _combine_body,
    out_type=jax.ShapeDtypeStruct((_N, _H), jnp.float32),
    mesh=_sc_mesh,
    compiler_params=_sc_params,
    scratch_types=[pltpu.VMEM((_NCH, 16), jnp.int32),
                   pltpu.VMEM((3, 16, _H), jnp.float32),
                   pltpu.SemaphoreType.DMA, pltpu.SemaphoreType.DMA,
                   pltpu.SemaphoreType.DMA, pltpu.SemaphoreType.DMA,
                   pltpu.SemaphoreType.DMA, pltpu.SemaphoreType.DMA],
)

_grouped_gemm = pl.pallas_call(
    _gemm_body,
    grid_spec=pltpu.PrefetchScalarGridSpec(
        num_scalar_prefetch=2,
        grid=(_NT,),
        in_specs=[
            pl.BlockSpec((_TILE, _H),
                         lambda t, te, na: (jnp.where(t < na[0], t, 0), 0)),
            pl.BlockSpec((1, _IE, _H), lambda t, te, na: (te[t], 0, 0)),
            pl.BlockSpec((1, _IE, _H), lambda t, te, na: (te[t], 0, 0)),
            pl.BlockSpec((1, _H, _IE), lambda t, te, na: (te[t], 0, 0)),
        ],
        out_specs=pl.BlockSpec(
            (_TILE, _H), lambda t, te, na: (jnp.where(t < na[0], t, _NT), 0)),
    ),
    out_shape=jax.ShapeDtypeStruct((_NS + _TILE, _H), jnp.float32),
)


def kernel(hidden_states, token_ids, gate_w, up_w, down_w):
    b, s, h = hidden_states.shape
    x = hidden_states.reshape(b * s, h)
    tids = token_ids.reshape(-1)
    counts, eid, rank = _route(tids)
    xs, pos, te, nact = _dispatch(x, counts, eid, rank)
    ys = _grouped_gemm(te, nact, xs,
                       gate_w.astype(jnp.bfloat16),
                       up_w.astype(jnp.bfloat16),
                       down_w.astype(jnp.bfloat16))
    out = _combine(ys, pos)
    return out.reshape(b, s, h)


# speedup vs baseline: 1.0999x; 1.0999x over previous
import functools

import jax
import jax.numpy as jnp
from jax import lax
from jax.experimental import pallas as pl
from jax.experimental.pallas import tpu as pltpu
from jax.experimental.pallas import tpu_sc as plsc

_V = 100000
_E = 64
_H = 2048
_IE = 32
_N = 8192
_NW = 32
_TPW = _N // _NW
_NCH = _TPW // 16
_TILE = 128
_NS = 16384
_NT = _NS // _TILE


def _wid():
    nc = plsc.get_sparse_core_info().num_cores
    return lax.axis_index("s") * nc + lax.axis_index("c")


def _route_body(tids_hbm, counts_hbm, eid_hbm, rank_hbm,
                tid_v, eid_v, rank_v, cnt_v):
    wid = _wid()
    base = wid * _TPW
    iota = lax.iota(jnp.int32, 16)
    pltpu.sync_copy(tids_hbm.at[pl.ds(base, _TPW)], tid_v)
    for c in range(_NCH):
        t = tid_v[pl.ds(c * 16, 16)]
        t = jnp.minimum(jnp.maximum(t, 0), _V - 1)
        eid_v[pl.ds(c * 16, 16)] = lax.rem(t, _E)
        rank_v[pl.ds(c * 16, 16)] = jnp.zeros((16,), jnp.int32)

    def e_body(e, carry_unused):
        ev = jnp.zeros((16,), jnp.int32) + e
        carry = jnp.int32(0)
        for c in range(_NCH):
            ee = eid_v[pl.ds(c * 16, 16)]
            m = ee == ev
            mm = m.astype(jnp.int32)
            exc = plsc.cumsum(mm) - mm
            old = rank_v[pl.ds(c * 16, 16)]
            rank_v[pl.ds(c * 16, 16)] = jnp.where(m, exc + carry, old)
            carry = carry + jnp.sum(mm)
        plsc.store_scatter(cnt_v, [ev], jnp.zeros((16,), jnp.int32) + carry,
                           mask=iota == 0)
        return carry_unused

    lax.fori_loop(0, _E, e_body, jnp.int32(0))
    pltpu.sync_copy(cnt_v, counts_hbm.at[wid])
    pltpu.sync_copy(eid_v, eid_hbm.at[wid])
    pltpu.sync_copy(rank_v, rank_hbm.at[wid])


def _dispatch_body(x_hbm, counts_hbm, eid_hbm, rank_hbm,
                   xs_hbm, pos_hbm, te_hbm, nact_hbm,
                   allcnt_v, start_v, eid_v, rank_v, slots_v, te_v, nact_v,
                   rowbuf_v, gs0, gs1, gs2, ss0, ss1, ss2):
    wid = _wid()
    base = wid * _TPW
    iota = lax.iota(jnp.int32, 16)
    pltpu.sync_copy(counts_hbm, allcnt_v)

    tot, myb = [], []
    for ec in range(_E // 16):
        t_acc = jnp.zeros((16,), jnp.int32)
        m_acc = jnp.zeros((16,), jnp.int32)
        for w in range(_NW):
            row = allcnt_v[w, pl.ds(ec * 16, 16)]
            t_acc = t_acc + row
            m_acc = m_acc + row * (wid > w).astype(jnp.int32)
        tot.append(t_acc)
        myb.append(m_acc)

    carry = jnp.int32(0)
    pend = []
    for ec in range(_E // 16):
        pc = (tot[ec] + (_TILE - 1)) & ~(_TILE - 1)
        exc = plsc.cumsum(pc) - pc
        ps = exc + carry
        pend.append(ps + pc)
        carry = carry + jnp.sum(pc)
        start_v[pl.ds(ec * 16, 16)] = ps + myb[ec]

    pltpu.sync_copy(eid_hbm.at[wid], eid_v)
    pltpu.sync_copy(rank_hbm.at[wid], rank_v)
    for c in range(_NCH):
        ee = eid_v[pl.ds(c * 16, 16)]
        st = plsc.load_gather(start_v, [ee])
        slots_v[c] = st + rank_v[pl.ds(c * 16, 16)]
    pltpu.sync_copy(slots_v, pos_hbm.at[wid])

    gsems = [gs0, gs1, gs2]
    ssems = [ss0, ss1, ss2]
    hg = [None] * _NCH
    hs = [None] * _NCH
    for c in range(3):
        hg[c] = pltpu.async_copy(x_hbm.at[pl.ds(base + c * 16, 16)],
                                 rowbuf_v.at[c], gsems[c])
    for c in range(_NCH):
        bslot = c % 3
        hg[c].wait()
        hs[c] = pltpu.async_copy(rowbuf_v.at[bslot], xs_hbm.at[slots_v.at[c]],
                                 ssems[bslot])
        nxt = c + 3
        if nxt < _NCH:
            hs[c].wait()
            hg[nxt] = pltpu.async_copy(x_hbm.at[pl.ds(base + nxt * 16, 16)],
                                       rowbuf_v.at[bslot], gsems[bslot])
    for c in range(_NCH - 3, _NCH):
        hs[c].wait()

    @pl.when(wid == 0)
    def _():
        ends = []
        for e in range(_E):
            ch, ln = e // 16, e % 16
            ends.append(jnp.sum(jnp.where(iota == ln, pend[ch], 0)))
        for tc in range(_NT // 16):
            tvec = (iota + tc * 16) * _TILE
            acc = jnp.zeros((16,), jnp.int32)
            for e in range(_E):
                acc = acc + (tvec >= ends[e]).astype(jnp.int32)
            te_v[pl.ds(tc * 16, 16)] = jnp.minimum(acc, _E - 1)
        pltpu.sync_copy(te_v, te_hbm)
        nact_v[...] = jnp.zeros((16,), jnp.int32) + lax.div(carry, _TILE)
        pltpu.sync_copy(nact_v, nact_hbm)


def _combine_body(ys_hbm, pos_hbm, out_hbm, pos_v, rowbuf_v,
                  gs0, gs1, gs2, ss0, ss1, ss2):
    wid = _wid()
    base = wid * _TPW
    pltpu.sync_copy(pos_hbm.at[wid], pos_v)
    gsems = [gs0, gs1, gs2]
    ssems = [ss0, ss1, ss2]
    hg = [None] * _NCH
    hs = [None] * _NCH
    for c in range(3):
        hg[c] = pltpu.async_copy(ys_hbm.at[pos_v.at[c]], rowbuf_v.at[c],
                                 gsems[c])
    for c in range(_NCH):
        bslot = c % 3
        hg[c].wait()
        hs[c] = pltpu.async_copy(rowbuf_v.at[bslot],
                                 out_hbm.at[pl.ds(base + c * 16, 16)],
                                 ssems[bslot])
        nxt = c + 3
        if nxt < _NCH:
            hs[c].wait()
            hg[nxt] = pltpu.async_copy(ys_hbm.at[pos_v.at[nxt]],
                                       rowbuf_v.at[bslot], gsems[bslot])
    for c in range(_NCH - 3, _NCH):
        hs[c].wait()


def _tobf16_body(x_ref, o_ref):
    o_ref[...] = x_ref[...].astype(jnp.bfloat16)


def _gemm_body(te_ref, na_ref, x_ref, g_ref, u_ref, d_ref, o_ref):
    e = te_ref[pl.program_id(0)]
    x = x_ref[...].astype(jnp.bfloat16)
    g = g_ref[e]
    u = u_ref[e]
    dn = d_ref[e]
    xg = lax.dot_general(x, g, (((1,), (1,)), ((), ())),
                         preferred_element_type=jnp.float32)
    xu = lax.dot_general(x, u, (((1,), (1,)), ((), ())),
                         preferred_element_type=jnp.float32)
    h = (xg * jax.nn.sigmoid(xg) * xu).astype(jnp.bfloat16)
    o_ref[...] = lax.dot_general(h, dn, (((1,), (1,)), ((), ())),
                                 preferred_element_type=jnp.float32)


_sc_mesh = plsc.VectorSubcoreMesh(core_axis_name="c", subcore_axis_name="s")
_sc_params = pltpu.CompilerParams(needs_layout_passes=False)

_route = pl.kernel(
    _route_body,
    out_type=(jax.ShapeDtypeStruct((_NW, _E), jnp.int32),
              jax.ShapeDtypeStruct((_NW, _TPW), jnp.int32),
              jax.ShapeDtypeStruct((_NW, _TPW), jnp.int32)),
    mesh=_sc_mesh,
    compiler_params=_sc_params,
    scratch_types=[pltpu.VMEM((_TPW,), jnp.int32),
                   pltpu.VMEM((_TPW,), jnp.int32),
                   pltpu.VMEM((_TPW,), jnp.int32),
                   pltpu.VMEM((_E,), jnp.int32)],
)

_tobf16 = pl.pallas_call(
    _tobf16_body,
    grid=(16,),
    in_specs=[pl.BlockSpec((_N // 16, _H), lambda t: (t, 0))],
    out_specs=pl.BlockSpec((_N // 16, _H), lambda t: (t, 0)),
    out_shape=jax.ShapeDtypeStruct((_N, _H), jnp.bfloat16),
)

_dispatch = pl.kernel(
    _dispatch_body,
    out_type=(jax.ShapeDtypeStruct((_NS, _H), jnp.float32),
              jax.ShapeDtypeStruct((_NW, _NCH, 16), jnp.int32),
              jax.ShapeDtypeStruct((_NT,), jnp.int32),
              jax.ShapeDtypeStruct((16,), jnp.int32)),
    mesh=_sc_mesh,
    compiler_params=_sc_params,
    scratch_types=[pltpu.VMEM((_NW, _E), jnp.int32),
                   pltpu.VMEM((_E,), jnp.int32),
                   pltpu.VMEM((_TPW,), jnp.int32),
                   pltpu.VMEM((_TPW,), jnp.int32),
                   pltpu.VMEM((_NCH, 16), jnp.int32),
                   pltpu.VMEM((_NT,), jnp.int32),
                   pltpu.VMEM((16,), jnp.int32),
                   pltpu.VMEM((3, 16, _H), jnp.float32),
                   pltpu.SemaphoreType.DMA, pltpu.SemaphoreType.DMA,
                   pltpu.SemaphoreType.DMA, pltpu.SemaphoreType.DMA,
                   pltpu.SemaphoreType.DMA, pltpu.SemaphoreType.DMA],
)

_combine = pl.kernel(
    _combine_body,
    out_type=jax.ShapeDtypeStruct((_N, _H), jnp.float32),
    mesh=_sc_mesh,
    compiler_params=_sc_params,
    scratch_types=[pltpu.VMEM((_NCH, 16), jnp.int32),
                   pltpu.VMEM((3, 16, _H), jnp.float32),
                   pltpu.SemaphoreType.DMA, pltpu.SemaphoreType.DMA,
                   pltpu.SemaphoreType.DMA, pltpu.SemaphoreType.DMA,
                   pltpu.SemaphoreType.DMA, pltpu.SemaphoreType.DMA],
)

_grouped_gemm = pl.pallas_call(
    _gemm_body,
    grid_spec=pltpu.PrefetchScalarGridSpec(
        num_scalar_prefetch=2,
        grid=(_NT,),
        in_specs=[
            pl.BlockSpec((_TILE, _H),
                         lambda t, te, na: (jnp.where(t < na[0], t, 0), 0)),
            pl.BlockSpec((_E, _IE, _H), lambda t, te, na: (0, 0, 0)),
            pl.BlockSpec((_E, _IE, _H), lambda t, te, na: (0, 0, 0)),
            pl.BlockSpec((_E, _H, _IE), lambda t, te, na: (0, 0, 0)),
        ],
        out_specs=pl.BlockSpec(
            (_TILE, _H), lambda t, te, na: (jnp.where(t < na[0], t, _NT), 0)),
    ),
    out_shape=jax.ShapeDtypeStruct((_NS + _TILE, _H), jnp.float32),
    compiler_params=pltpu.CompilerParams(vmem_limit_bytes=56 * 1024 * 1024),
)


def kernel(hidden_states, token_ids, gate_w, up_w, down_w):
    b, s, h = hidden_states.shape
    x = hidden_states.reshape(b * s, h)
    tids = token_ids.reshape(-1)
    counts, eid, rank = _route(tids)
    xs, pos, te, nact = _dispatch(x, counts, eid, rank)
    ys = _grouped_gemm(te, nact, xs,
                       gate_w.astype(jnp.bfloat16),
                       up_w.astype(jnp.bfloat16),
                       down_w.astype(jnp.bfloat16))
    out = _combine(ys, pos)
    return out.reshape(b, s, h)


# speedup vs baseline: 1.2089x; 1.0991x over previous
import functools

import jax
import jax.numpy as jnp
from jax import lax
from jax.experimental import pallas as pl
from jax.experimental.pallas import tpu as pltpu
from jax.experimental.pallas import tpu_sc as plsc

_V = 100000
_E = 64
_H = 2048
_IE = 32
_N = 8192
_NW = 32
_TPW = _N // _NW
_NCH = _TPW // 16
_TILE = 128
_NS = 16384
_NT = _NS // _TILE


def _wid():
    nc = plsc.get_sparse_core_info().num_cores
    return lax.axis_index("s") * nc + lax.axis_index("c")


def _route_body(tids_hbm, counts_hbm, eid_hbm, rank_hbm,
                tid_v, eid_v, rank_v, cnt_v):
    wid = _wid()
    base = wid * _TPW
    iota = lax.iota(jnp.int32, 16)
    pltpu.sync_copy(tids_hbm.at[pl.ds(base, _TPW)], tid_v)
    for c in range(_NCH):
        t = tid_v[pl.ds(c * 16, 16)]
        t = jnp.minimum(jnp.maximum(t, 0), _V - 1)
        eid_v[pl.ds(c * 16, 16)] = lax.rem(t, _E)
        rank_v[pl.ds(c * 16, 16)] = jnp.zeros((16,), jnp.int32)

    def e_body(e, carry_unused):
        ev = jnp.zeros((16,), jnp.int32) + e
        carry = jnp.int32(0)
        for c in range(_NCH):
            ee = eid_v[pl.ds(c * 16, 16)]
            m = ee == ev
            mm = m.astype(jnp.int32)
            exc = plsc.cumsum(mm) - mm
            old = rank_v[pl.ds(c * 16, 16)]
            rank_v[pl.ds(c * 16, 16)] = jnp.where(m, exc + carry, old)
            carry = carry + jnp.sum(mm)
        plsc.store_scatter(cnt_v, [ev], jnp.zeros((16,), jnp.int32) + carry,
                           mask=iota == 0)
        return carry_unused

    lax.fori_loop(0, _E, e_body, jnp.int32(0))
    pltpu.sync_copy(cnt_v, counts_hbm.at[wid])
    pltpu.sync_copy(eid_v, eid_hbm.at[wid])
    pltpu.sync_copy(rank_v, rank_hbm.at[wid])


def _dispatch_body(x_hbm, counts_hbm, eid_hbm, rank_hbm,
                   xs_hbm, pos_hbm, te_hbm, nact_hbm,
                   allcnt_v, start_v, eid_v, rank_v, slots_v, te_v, nact_v,
                   rowbuf_v, gs0, gs1, gs2, ss0, ss1, ss2):
    wid = _wid()
    base = wid * _TPW
    iota = lax.iota(jnp.int32, 16)
    pltpu.sync_copy(counts_hbm, allcnt_v)

    tot, myb = [], []
    for ec in range(_E // 16):
        t_acc = jnp.zeros((16,), jnp.int32)
        m_acc = jnp.zeros((16,), jnp.int32)
        for w in range(_NW):
            row = allcnt_v[w, pl.ds(ec * 16, 16)]
            t_acc = t_acc + row
            m_acc = m_acc + row * (wid > w).astype(jnp.int32)
        tot.append(t_acc)
        myb.append(m_acc)

    carry = jnp.int32(0)
    pend = []
    for ec in range(_E // 16):
        pc = (tot[ec] + (_TILE - 1)) & ~(_TILE - 1)
        exc = plsc.cumsum(pc) - pc
        ps = exc + carry
        pend.append(ps + pc)
        carry = carry + jnp.sum(pc)
        start_v[pl.ds(ec * 16, 16)] = ps + myb[ec]

    pltpu.sync_copy(eid_hbm.at[wid], eid_v)
    pltpu.sync_copy(rank_hbm.at[wid], rank_v)
    for c in range(_NCH):
        ee = eid_v[pl.ds(c * 16, 16)]
        st = plsc.load_gather(start_v, [ee])
        slots_v[c] = st + rank_v[pl.ds(c * 16, 16)]
    pltpu.sync_copy(slots_v, pos_hbm.at[wid])

    gsems = [gs0, gs1, gs2]
    ssems = [ss0, ss1, ss2]
    hg = [None] * _NCH
    hs = [None] * _NCH
    for c in range(3):
        hg[c] = pltpu.async_copy(x_hbm.at[pl.ds(base + c * 16, 16)],
                                 rowbuf_v.at[c], gsems[c])
    for c in range(_NCH):
        bslot = c % 3
        hg[c].wait()
        hs[c] = pltpu.async_copy(rowbuf_v.at[bslot], xs_hbm.at[slots_v.at[c]],
                                 ssems[bslot])
        nxt = c + 3
        if nxt < _NCH:
            hs[c].wait()
            hg[nxt] = pltpu.async_copy(x_hbm.at[pl.ds(base + nxt * 16, 16)],
                                       rowbuf_v.at[bslot], gsems[bslot])
    for c in range(_NCH - 3, _NCH):
        hs[c].wait()

    @pl.when(wid == 0)
    def _():
        ends = []
        for e in range(_E):
            ch, ln = e // 16, e % 16
            ends.append(jnp.sum(jnp.where(iota == ln, pend[ch], 0)))
        for tc in range(_NT // 16):
            tvec = (iota + tc * 16) * _TILE
            acc = jnp.zeros((16,), jnp.int32)
            for e in range(_E):
                acc = acc + (tvec >= ends[e]).astype(jnp.int32)
            te_v[pl.ds(tc * 16, 16)] = jnp.minimum(acc, _E - 1)
        pltpu.sync_copy(te_v, te_hbm)
        nact_v[...] = jnp.zeros((16,), jnp.int32) + lax.div(carry + 255, 256)
        pltpu.sync_copy(nact_v, nact_hbm)


def _combine_body(ys_hbm, pos_hbm, out_hbm, pos_v, rowbuf_v,
                  gs0, gs1, gs2, ss0, ss1, ss2):
    wid = _wid()
    base = wid * _TPW
    pltpu.sync_copy(pos_hbm.at[wid], pos_v)
    gsems = [gs0, gs1, gs2]
    ssems = [ss0, ss1, ss2]
    hg = [None] * _NCH
    hs = [None] * _NCH
    for c in range(3):
        hg[c] = pltpu.async_copy(ys_hbm.at[pos_v.at[c]], rowbuf_v.at[c],
                                 gsems[c])
    for c in range(_NCH):
        bslot = c % 3
        hg[c].wait()
        hs[c] = pltpu.async_copy(rowbuf_v.at[bslot],
                                 out_hbm.at[pl.ds(base + c * 16, 16)],
                                 ssems[bslot])
        nxt = c + 3
        if nxt < _NCH:
            hs[c].wait()
            hg[nxt] = pltpu.async_copy(ys_hbm.at[pos_v.at[nxt]],
                                       rowbuf_v.at[bslot], gsems[bslot])
    for c in range(_NCH - 3, _NCH):
        hs[c].wait()


def _tobf16_body(x_ref, o_ref):
    o_ref[...] = x_ref[...].astype(jnp.bfloat16)


_SUB = 2
_BLK = _SUB * _TILE
_NB = _NS // _BLK


def _gemm_body(te_ref, na_ref, x_ref, g_ref, u_ref, d_ref, o_ref):
    t = pl.program_id(0)
    for j in range(_SUB):
        e = te_ref[t * _SUB + j]
        x = x_ref[pl.ds(j * _TILE, _TILE), :].astype(jnp.bfloat16)
        g = g_ref[e]
        u = u_ref[e]
        dn = d_ref[e]
        xg = lax.dot_general(x, g, (((1,), (1,)), ((), ())),
                             preferred_element_type=jnp.float32)
        xu = lax.dot_general(x, u, (((1,), (1,)), ((), ())),
                             preferred_element_type=jnp.float32)
        h = (xg * jax.nn.sigmoid(xg) * xu).astype(jnp.bfloat16)
        o_ref[pl.ds(j * _TILE, _TILE), :] = lax.dot_general(
            h, dn, (((1,), (1,)), ((), ())),
            preferred_element_type=jnp.float32)


_sc_mesh = plsc.VectorSubcoreMesh(core_axis_name="c", subcore_axis_name="s")
_sc_params = pltpu.CompilerParams(needs_layout_passes=False)

_route = pl.kernel(
    _route_body,
    out_type=(jax.ShapeDtypeStruct((_NW, _E), jnp.int32),
              jax.ShapeDtypeStruct((_NW, _TPW), jnp.int32),
              jax.ShapeDtypeStruct((_NW, _TPW), jnp.int32)),
    mesh=_sc_mesh,
    compiler_params=_sc_params,
    scratch_types=[pltpu.VMEM((_TPW,), jnp.int32),
                   pltpu.VMEM((_TPW,), jnp.int32),
                   pltpu.VMEM((_TPW,), jnp.int32),
                   pltpu.VMEM((_E,), jnp.int32)],
)

_tobf16 = pl.pallas_call(
    _tobf16_body,
    grid=(16,),
    in_specs=[pl.BlockSpec((_N // 16, _H), lambda t: (t, 0))],
    out_specs=pl.BlockSpec((_N // 16, _H), lambda t: (t, 0)),
    out_shape=jax.ShapeDtypeStruct((_N, _H), jnp.bfloat16),
)

_dispatch = pl.kernel(
    _dispatch_body,
    out_type=(jax.ShapeDtypeStruct((_NS, _H), jnp.float32),
              jax.ShapeDtypeStruct((_NW, _NCH, 16), jnp.int32),
              jax.ShapeDtypeStruct((_NT,), jnp.int32),
              jax.ShapeDtypeStruct((16,), jnp.int32)),
    mesh=_sc_mesh,
    compiler_params=_sc_params,
    scratch_types=[pltpu.VMEM((_NW, _E), jnp.int32),
                   pltpu.VMEM((_E,), jnp.int32),
                   pltpu.VMEM((_TPW,), jnp.int32),
                   pltpu.VMEM((_TPW,), jnp.int32),
                   pltpu.VMEM((_NCH, 16), jnp.int32),
                   pltpu.VMEM((_NT,), jnp.int32),
                   pltpu.VMEM((16,), jnp.int32),
                   pltpu.VMEM((3, 16, _H), jnp.float32),
                   pltpu.SemaphoreType.DMA, pltpu.SemaphoreType.DMA,
                   pltpu.SemaphoreType.DMA, pltpu.SemaphoreType.DMA,
                   pltpu.SemaphoreType.DMA, pltpu.SemaphoreType.DMA],
)

_combine = pl.kernel(
    _combine_body,
    out_type=jax.ShapeDtypeStruct((_N, _H), jnp.float32),
    mesh=_sc_mesh,
    compiler_params=_sc_params,
    scratch_types=[pltpu.VMEM((_NCH, 16), jnp.int32),
                   pltpu.VMEM((3, 16, _H), jnp.float32),
                   pltpu.SemaphoreType.DMA, pltpu.SemaphoreType.DMA,
                   pltpu.SemaphoreType.DMA, pltpu.SemaphoreType.DMA,
                   pltpu.SemaphoreType.DMA, pltpu.SemaphoreType.DMA],
)

_grouped_gemm = pl.pallas_call(
    _gemm_body,
    grid_spec=pltpu.PrefetchScalarGridSpec(
        num_scalar_prefetch=2,
        grid=(_NB,),
        in_specs=[
            pl.BlockSpec((_BLK, _H),
                         lambda t, te, na: (jnp.where(t < na[0], t, 0), 0)),
            pl.BlockSpec((_E, _IE, _H), lambda t, te, na: (0, 0, 0)),
            pl.BlockSpec((_E, _IE, _H), lambda t, te, na: (0, 0, 0)),
            pl.BlockSpec((_E, _H, _IE), lambda t, te, na: (0, 0, 0)),
        ],
        out_specs=pl.BlockSpec(
            (_BLK, _H), lambda t, te, na: (jnp.where(t < na[0], t, _NB), 0)),
    ),
    out_shape=jax.ShapeDtypeStruct((_NS + _BLK, _H), jnp.float32),
    compiler_params=pltpu.CompilerParams(vmem_limit_bytes=56 * 1024 * 1024),
)


def kernel(hidden_states, token_ids, gate_w, up_w, down_w):
    b, s, h = hidden_states.shape
    x = hidden_states.reshape(b * s, h)
    tids = token_ids.reshape(-1)
    counts, eid, rank = _route(tids)
    xs, pos, te, nact = _dispatch(x, counts, eid, rank)
    ys = _grouped_gemm(te, nact, xs,
                       gate_w.astype(jnp.bfloat16),
                       up_w.astype(jnp.bfloat16),
                       down_w.astype(jnp.bfloat16))
    out = _combine(ys, pos)
    return out.reshape(b, s, h)


# speedup vs baseline: 1.3737x; 1.1363x over previous
import functools

import jax
import jax.numpy as jnp
from jax import lax
from jax.experimental import pallas as pl
from jax.experimental.pallas import tpu as pltpu
from jax.experimental.pallas import tpu_sc as plsc

_V = 100000
_E = 64
_H = 2048
_IE = 32
_N = 8192
_NW = 32
_TPW = _N // _NW
_NCH = _TPW // 16
_TILE = 128
_NS = 16384
_NT = _NS // _TILE


def _wid():
    nc = plsc.get_sparse_core_info().num_cores
    return lax.axis_index("s") * nc + lax.axis_index("c")


def _route_body(tids_hbm, counts_hbm, eid_hbm, rank_hbm,
                tid_v, eid_v, rank_v, cnt_v):
    wid = _wid()
    base = wid * _TPW
    iota = lax.iota(jnp.int32, 16)
    pltpu.sync_copy(tids_hbm.at[pl.ds(base, _TPW)], tid_v)
    for c in range(_NCH):
        t = tid_v[pl.ds(c * 16, 16)]
        t = jnp.minimum(jnp.maximum(t, 0), _V - 1)
        eid_v[pl.ds(c * 16, 16)] = lax.rem(t, _E)
        rank_v[pl.ds(c * 16, 16)] = jnp.zeros((16,), jnp.int32)

    def e_body(e, carry_unused):
        ev = jnp.zeros((16,), jnp.int32) + e
        carry = jnp.int32(0)
        for c in range(_NCH):
            ee = eid_v[pl.ds(c * 16, 16)]
            m = ee == ev
            mm = m.astype(jnp.int32)
            exc = plsc.cumsum(mm) - mm
            old = rank_v[pl.ds(c * 16, 16)]
            rank_v[pl.ds(c * 16, 16)] = jnp.where(m, exc + carry, old)
            carry = carry + jnp.sum(mm)
        plsc.store_scatter(cnt_v, [ev], jnp.zeros((16,), jnp.int32) + carry,
                           mask=iota == 0)
        return carry_unused

    lax.fori_loop(0, _E, e_body, jnp.int32(0))
    pltpu.sync_copy(cnt_v, counts_hbm.at[wid])
    pltpu.sync_copy(eid_v, eid_hbm.at[wid])
    pltpu.sync_copy(rank_v, rank_hbm.at[wid])


def _dispatch_body(x_hbm, counts_hbm, eid_hbm, rank_hbm,
                   xs_hbm, pos_hbm, te_hbm, nact_hbm,
                   allcnt_v, start_v, eid_v, rank_v, slots_v, te_v, nact_v,
                   rowbuf_v, gs0, gs1, gs2, ss0, ss1, ss2):
    wid = _wid()
    base = wid * _TPW
    iota = lax.iota(jnp.int32, 16)
    pltpu.sync_copy(counts_hbm, allcnt_v)

    tot, myb = [], []
    for ec in range(_E // 16):
        t_acc = jnp.zeros((16,), jnp.int32)
        m_acc = jnp.zeros((16,), jnp.int32)
        for w in range(_NW):
            row = allcnt_v[w, pl.ds(ec * 16, 16)]
            t_acc = t_acc + row
            m_acc = m_acc + row * (wid > w).astype(jnp.int32)
        tot.append(t_acc)
        myb.append(m_acc)

    carry = jnp.int32(0)
    pend = []
    for ec in range(_E // 16):
        pc = (tot[ec] + (_TILE - 1)) & ~(_TILE - 1)
        exc = plsc.cumsum(pc) - pc
        ps = exc + carry
        pend.append(ps + pc)
        carry = carry + jnp.sum(pc)
        start_v[pl.ds(ec * 16, 16)] = ps + myb[ec]

    pltpu.sync_copy(eid_hbm.at[wid], eid_v)
    pltpu.sync_copy(rank_hbm.at[wid], rank_v)
    for c in range(_NCH):
        ee = eid_v[pl.ds(c * 16, 16)]
        st = plsc.load_gather(start_v, [ee])
        slots_v[c] = st + rank_v[pl.ds(c * 16, 16)]
    pltpu.sync_copy(slots_v, pos_hbm.at[wid])

    gsems = [gs0, gs1, gs2]
    ssems = [ss0, ss1, ss2]
    hg = [None] * _NCH
    hs = [None] * _NCH
    for c in range(3):
        hg[c] = pltpu.async_copy(x_hbm.at[pl.ds(base + c * 16, 16)],
                                 rowbuf_v.at[c], gsems[c])
    for c in range(_NCH):
        bslot = c % 3
        hg[c].wait()
        hs[c] = pltpu.async_copy(rowbuf_v.at[bslot], xs_hbm.at[slots_v.at[c]],
                                 ssems[bslot])
        nxt = c + 3
        if nxt < _NCH:
            hs[c].wait()
            hg[nxt] = pltpu.async_copy(x_hbm.at[pl.ds(base + nxt * 16, 16)],
                                       rowbuf_v.at[bslot], gsems[bslot])
    for c in range(_NCH - 3, _NCH):
        hs[c].wait()

    @pl.when(wid == 0)
    def _():
        ends = []
        for e in range(_E):
            ch, ln = e // 16, e % 16
            ends.append(jnp.sum(jnp.where(iota == ln, pend[ch], 0)))
        for tc in range(_NT // 16):
            tvec = (iota + tc * 16) * _TILE
            acc = jnp.zeros((16,), jnp.int32)
            for e in range(_E):
                acc = acc + (tvec >= ends[e]).astype(jnp.int32)
            te_v[pl.ds(tc * 16, 16)] = jnp.minimum(acc, _E - 1)
        pltpu.sync_copy(te_v, te_hbm)
        nact_v[...] = jnp.zeros((16,), jnp.int32) + lax.div(carry + 511, 512)
        pltpu.sync_copy(nact_v, nact_hbm)


def _combine_body(ys_hbm, pos_hbm, out_hbm, pos_v, rowbuf_v,
                  gs0, gs1, gs2, ss0, ss1, ss2):
    wid = _wid()
    base = wid * _TPW
    pltpu.sync_copy(pos_hbm.at[wid], pos_v)
    gsems = [gs0, gs1, gs2]
    ssems = [ss0, ss1, ss2]
    hg = [None] * _NCH
    hs = [None] * _NCH
    for c in range(3):
        hg[c] = pltpu.async_copy(ys_hbm.at[pos_v.at[c]], rowbuf_v.at[c],
                                 gsems[c])
    for c in range(_NCH):
        bslot = c % 3
        hg[c].wait()
        hs[c] = pltpu.async_copy(rowbuf_v.at[bslot],
                                 out_hbm.at[pl.ds(base + c * 16, 16)],
                                 ssems[bslot])
        nxt = c + 3
        if nxt < _NCH:
            hs[c].wait()
            hg[nxt] = pltpu.async_copy(ys_hbm.at[pos_v.at[nxt]],
                                       rowbuf_v.at[bslot], gsems[bslot])
    for c in range(_NCH - 3, _NCH):
        hs[c].wait()


def _tobf16_body(x_ref, o_ref):
    o_ref[...] = x_ref[...].astype(jnp.bfloat16)


_SUB = 4
_BLK = _SUB * _TILE
_NB = _NS // _BLK


def _gemm_body(te_ref, na_ref, x_ref, g_hbm, u_hbm, d_hbm, o_ref,
               g_ref, u_ref, d_ref, wsem):
    t = pl.program_id(0)

    @pl.when(t == 0)
    def _():
        pltpu.make_async_copy(g_hbm, g_ref, wsem).start()
        pltpu.make_async_copy(g_hbm, g_ref, wsem).wait()
        pltpu.make_async_copy(u_hbm, u_ref, wsem).start()
        pltpu.make_async_copy(u_hbm, u_ref, wsem).wait()
        pltpu.make_async_copy(d_hbm, d_ref, wsem).start()
        pltpu.make_async_copy(d_hbm, d_ref, wsem).wait()

    for j in range(_SUB):
        e = te_ref[t * _SUB + j]
        x = x_ref[pl.ds(j * _TILE, _TILE), :].astype(jnp.bfloat16)
        g = g_ref[e]
        u = u_ref[e]
        dn = d_ref[e]
        xg = lax.dot_general(x, g, (((1,), (1,)), ((), ())),
                             preferred_element_type=jnp.float32)
        xu = lax.dot_general(x, u, (((1,), (1,)), ((), ())),
                             preferred_element_type=jnp.float32)
        h = (xg * jax.nn.sigmoid(xg) * xu).astype(jnp.bfloat16)
        o_ref[pl.ds(j * _TILE, _TILE), :] = lax.dot_general(
            h, dn, (((1,), (0,)), ((), ())),
            preferred_element_type=jnp.float32)


_sc_mesh = plsc.VectorSubcoreMesh(core_axis_name="c", subcore_axis_name="s")
_sc_params = pltpu.CompilerParams(needs_layout_passes=False)

_route = pl.kernel(
    _route_body,
    out_type=(jax.ShapeDtypeStruct((_NW, _E), jnp.int32),
              jax.ShapeDtypeStruct((_NW, _TPW), jnp.int32),
              jax.ShapeDtypeStruct((_NW, _TPW), jnp.int32)),
    mesh=_sc_mesh,
    compiler_params=_sc_params,
    scratch_types=[pltpu.VMEM((_TPW,), jnp.int32),
                   pltpu.VMEM((_TPW,), jnp.int32),
                   pltpu.VMEM((_TPW,), jnp.int32),
                   pltpu.VMEM((_E,), jnp.int32)],
)

_tobf16 = pl.pallas_call(
    _tobf16_body,
    grid=(16,),
    in_specs=[pl.BlockSpec((_N // 16, _H), lambda t: (t, 0))],
    out_specs=pl.BlockSpec((_N // 16, _H), lambda t: (t, 0)),
    out_shape=jax.ShapeDtypeStruct((_N, _H), jnp.bfloat16),
)

_dispatch = pl.kernel(
    _dispatch_body,
    out_type=(jax.ShapeDtypeStruct((_NS, _H), jnp.float32),
              jax.ShapeDtypeStruct((_NW, _NCH, 16), jnp.int32),
              jax.ShapeDtypeStruct((_NT,), jnp.int32),
              jax.ShapeDtypeStruct((16,), jnp.int32)),
    mesh=_sc_mesh,
    compiler_params=_sc_params,
    scratch_types=[pltpu.VMEM((_NW, _E), jnp.int32),
                   pltpu.VMEM((_E,), jnp.int32),
                   pltpu.VMEM((_TPW,), jnp.int32),
                   pltpu.VMEM((_TPW,), jnp.int32),
                   pltpu.VMEM((_NCH, 16), jnp.int32),
                   pltpu.VMEM((_NT,), jnp.int32),
                   pltpu.VMEM((16,), jnp.int32),
                   pltpu.VMEM((3, 16, _H), jnp.float32),
                   pltpu.SemaphoreType.DMA, pltpu.SemaphoreType.DMA,
                   pltpu.SemaphoreType.DMA, pltpu.SemaphoreType.DMA,
                   pltpu.SemaphoreType.DMA, pltpu.SemaphoreType.DMA],
)

_combine = pl.kernel(
    _combine_body,
    out_type=jax.ShapeDtypeStruct((_N, _H), jnp.float32),
    mesh=_sc_mesh,
    compiler_params=_sc_params,
    scratch_types=[pltpu.VMEM((_NCH, 16), jnp.int32),
                   pltpu.VMEM((3, 16, _H), jnp.float32),
                   pltpu.SemaphoreType.DMA, pltpu.SemaphoreType.DMA,
                   pltpu.SemaphoreType.DMA, pltpu.SemaphoreType.DMA,
                   pltpu.SemaphoreType.DMA, pltpu.SemaphoreType.DMA],
)

_grouped_gemm = pl.pallas_call(
    _gemm_body,
    grid_spec=pltpu.PrefetchScalarGridSpec(
        num_scalar_prefetch=2,
        grid=(_NB,),
        in_specs=[
            pl.BlockSpec((_BLK, _H),
                         lambda t, te, na: (jnp.where(t < na[0], t, 0), 0)),
            pl.BlockSpec(memory_space=pl.ANY),
            pl.BlockSpec(memory_space=pl.ANY),
            pl.BlockSpec(memory_space=pl.ANY),
        ],
        out_specs=pl.BlockSpec(
            (_BLK, _H),
            lambda t, te, na: (jnp.where(t < na[0], t,
                                         jnp.minimum(na[0], _NB - 1)), 0)),
        scratch_shapes=[pltpu.VMEM((_E, _IE, _H), jnp.bfloat16),
                        pltpu.VMEM((_E, _IE, _H), jnp.bfloat16),
                        pltpu.VMEM((_E, _IE, _H), jnp.bfloat16),
                        pltpu.SemaphoreType.DMA],
    ),
    out_shape=jax.ShapeDtypeStruct((_NS, _H), jnp.float32),
    compiler_params=pltpu.CompilerParams(vmem_limit_bytes=56 * 1024 * 1024),
)


def kernel(hidden_states, token_ids, gate_w, up_w, down_w):
    b, s, h = hidden_states.shape
    x = hidden_states.reshape(b * s, h)
    tids = token_ids.reshape(-1)
    counts, eid, rank = _route(tids)
    xs, pos, te, nact = _dispatch(x, counts, eid, rank)
    ys = _grouped_gemm(te, nact, xs,
                       gate_w.astype(jnp.bfloat16),
                       up_w.astype(jnp.bfloat16),
                       jnp.swapaxes(down_w, 1, 2).astype(jnp.bfloat16))
    out = _combine(ys, pos)
    return out.reshape(b, s, h)
